# baseline (speedup 1.0000x reference)
import functools

import jax
import jax.numpy as jnp
from jax.experimental import pallas as pl
from jax.experimental.pallas import tpu as pltpu

BATCH, V, SAMPLES = 32, 64, 4096
EMBED_DIM = 128
WINDOW_SIZE = 64
STEP_SIZE = 32
NUM_TOKENS = (SAMPLES - WINDOW_SIZE) // STEP_SIZE
NUM_CHUNKS = SAMPLES // STEP_SIZE
CHUNK_FLAT = STEP_SIZE * EMBED_DIM
ROWS = 2
GRID = BATCH // ROWS


def _split_hi_lo(a):
    bits = jax.lax.bitcast_convert_type(a, jnp.uint32)
    hi = jax.lax.bitcast_convert_type(
        jax.lax.bitwise_and(bits, jnp.uint32(0xFFFF0000)), jnp.float32)
    return hi.astype(jnp.bfloat16), (a - hi).astype(jnp.bfloat16)


def _fused_kernel(x_ref, w1_ref, b1_ref, wcat_ref, b2_ref, out_ref,
                  scr_a, scr_b):
    i = pl.program_id(0)

    def stage1_into(scr):
        xh, xl = _split_hi_lo(x_ref[...])
        cols = [jnp.concatenate([xh[r], xl[r], xh[r], xl[r]], axis=0)
                for r in range(ROWS)]
        lhs = jnp.concatenate(cols, axis=1)
        proj = jax.lax.dot_general(
            lhs, w1_ref[...],
            dimension_numbers=(((0,), (0,)), ((), ())),
            preferred_element_type=jnp.float32)
        ns = jnp.floor(proj + b1_ref[...][None, :])
        scr[...] = ns.astype(jnp.bfloat16).reshape(ROWS * NUM_CHUNKS,
                                                   CHUNK_FLAT)

    def stage2_from(scr):
        r = jnp.dot(scr[...], wcat_ref[...],
                    preferred_element_type=jnp.float32)
        u = r[:, :EMBED_DIM]
        v = r[:, EMBED_DIM:]
        b2v = b2_ref[...][None, :]
        for r_i in range(ROWS):
            base = r_i * NUM_CHUNKS
            out_ref[r_i] = jnp.floor(
                u[base:base + NUM_TOKENS]
                + v[base + 1:base + NUM_TOKENS + 1] + b2v)

    even = jax.lax.bitwise_and(i, jnp.int32(1)) == jnp.int32(0)

    @pl.when(even)
    def _():
        stage1_into(scr_a)
        stage2_from(scr_b)

    @pl.when(jnp.logical_not(even))
    def _():
        stage1_into(scr_b)
        stage2_from(scr_a)


def _imap_x(i):
    z = i - i
    return (jnp.minimum(i, GRID - 1), z, z)


def _imap_out(i):
    z = i - i
    return (jnp.maximum(i - 1, 0), z, z)


def _imap2(i):
    z = i - i
    return (z, z)


def _imap1(i):
    return (i - i,)


@functools.partial(jax.jit, static_argnames=())
def kernel(x, W1, b1, W2, b2):
    xf = x.astype(jnp.float32)
    w1f = W1.astype(jnp.float32)
    b1f = b1.astype(jnp.float32)
    b2f = b2.astype(jnp.float32)
    w1bits = jax.lax.bitcast_convert_type(w1f, jnp.uint32)
    w1hf = jax.lax.bitcast_convert_type(
        jax.lax.bitwise_and(w1bits, jnp.uint32(0xFFFF0000)), jnp.float32)
    w1h = w1hf.astype(jnp.bfloat16)
    w1l = (w1f - w1hf).astype(jnp.bfloat16)
    w1cat = jnp.concatenate([w1h, w1h, w1l, w1l], axis=0)
    wcat = (W2.astype(jnp.bfloat16)
            .reshape(EMBED_DIM, 2, STEP_SIZE, EMBED_DIM)
            .transpose(2, 0, 1, 3)
            .reshape(CHUNK_FLAT, 2 * EMBED_DIM))

    tokens = pl.pallas_call(
        _fused_kernel,
        grid=(GRID + 1,),
        in_specs=[
            pl.BlockSpec((ROWS, V, SAMPLES), _imap_x),
            pl.BlockSpec((4 * V, EMBED_DIM), _imap2),
            pl.BlockSpec((EMBED_DIM,), _imap1),
            pl.BlockSpec((CHUNK_FLAT, 2 * EMBED_DIM), _imap2),
            pl.BlockSpec((EMBED_DIM,), _imap1),
        ],
        out_specs=pl.BlockSpec((ROWS, NUM_TOKENS, EMBED_DIM), _imap_out),
        out_shape=jax.ShapeDtypeStruct((BATCH, NUM_TOKENS, EMBED_DIM),
                                       jnp.float32),
        scratch_shapes=[
            pltpu.VMEM((ROWS * NUM_CHUNKS, CHUNK_FLAT), jnp.bfloat16),
            pltpu.VMEM((ROWS * NUM_CHUNKS, CHUNK_FLAT), jnp.bfloat16)],
    )(xf, w1cat, b1f, wcat, b2f)

    return tokens.astype(jnp.float64)


# speedup vs baseline: 1.0124x; 1.0124x over previous
import functools

import jax
import jax.numpy as jnp
from jax.experimental import pallas as pl
from jax.experimental.pallas import tpu as pltpu

BATCH, V, SAMPLES = 32, 64, 4096
EMBED_DIM = 128
WINDOW_SIZE = 64
STEP_SIZE = 32
NUM_TOKENS = (SAMPLES - WINDOW_SIZE) // STEP_SIZE
NUM_CHUNKS = SAMPLES // STEP_SIZE
CHUNK_FLAT = STEP_SIZE * EMBED_DIM
ROWS = 4
GRID = BATCH // ROWS


def _split_hi_lo(a):
    bits = jax.lax.bitcast_convert_type(a, jnp.uint32)
    hi = jax.lax.bitcast_convert_type(
        jax.lax.bitwise_and(bits, jnp.uint32(0xFFFF0000)), jnp.float32)
    return hi.astype(jnp.bfloat16), (a - hi).astype(jnp.bfloat16)


def _fused_kernel(x_ref, w1_ref, b1_ref, wcat_ref, b2_ref, out_ref,
                  scr_a, scr_b):
    i = pl.program_id(0)

    def stage1_into(scr):
        xh, xl = _split_hi_lo(x_ref[...])
        cols = [jnp.concatenate([xh[r], xl[r], xh[r], xl[r]], axis=0)
                for r in range(ROWS)]
        lhs = jnp.concatenate(cols, axis=1)
        proj = jax.lax.dot_general(
            lhs, w1_ref[...],
            dimension_numbers=(((0,), (0,)), ((), ())),
            preferred_element_type=jnp.float32)
        ns = jnp.floor(proj + b1_ref[...][None, :])
        scr[...] = ns.astype(jnp.bfloat16).reshape(ROWS * NUM_CHUNKS,
                                                   CHUNK_FLAT)

    def stage2_from(scr):
        r = jnp.dot(scr[...], wcat_ref[...],
                    preferred_element_type=jnp.float32)
        u = r[:, :EMBED_DIM]
        v = r[:, EMBED_DIM:]
        b2v = b2_ref[...][None, :]
        for r_i in range(ROWS):
            base = r_i * NUM_CHUNKS
            out_ref[r_i] = jnp.floor(
                u[base:base + NUM_TOKENS]
                + v[base + 1:base + NUM_TOKENS + 1] + b2v)

    even = jax.lax.bitwise_and(i, jnp.int32(1)) == jnp.int32(0)

    @pl.when(even)
    def _():
        stage1_into(scr_a)
        stage2_from(scr_b)

    @pl.when(jnp.logical_not(even))
    def _():
        stage1_into(scr_b)
        stage2_from(scr_a)


def _imap_x(i):
    z = i - i
    return (jnp.minimum(i, GRID - 1), z, z)


def _imap_out(i):
    z = i - i
    return (jnp.maximum(i - 1, 0), z, z)


def _imap2(i):
    z = i - i
    return (z, z)


def _imap1(i):
    return (i - i,)


@functools.partial(jax.jit, static_argnames=())
def kernel(x, W1, b1, W2, b2):
    xf = x.astype(jnp.float32)
    w1f = W1.astype(jnp.float32)
    b1f = b1.astype(jnp.float32)
    b2f = b2.astype(jnp.float32)
    w1bits = jax.lax.bitcast_convert_type(w1f, jnp.uint32)
    w1hf = jax.lax.bitcast_convert_type(
        jax.lax.bitwise_and(w1bits, jnp.uint32(0xFFFF0000)), jnp.float32)
    w1h = w1hf.astype(jnp.bfloat16)
    w1l = (w1f - w1hf).astype(jnp.bfloat16)
    w1cat = jnp.concatenate([w1h, w1h, w1l, w1l], axis=0)
    wcat = (W2.astype(jnp.bfloat16)
            .reshape(EMBED_DIM, 2, STEP_SIZE, EMBED_DIM)
            .transpose(2, 0, 1, 3)
            .reshape(CHUNK_FLAT, 2 * EMBED_DIM))

    tokens = pl.pallas_call(
        _fused_kernel,
        grid=(GRID + 1,),
        in_specs=[
            pl.BlockSpec((ROWS, V, SAMPLES), _imap_x),
            pl.BlockSpec((4 * V, EMBED_DIM), _imap2),
            pl.BlockSpec((EMBED_DIM,), _imap1),
            pl.BlockSpec((CHUNK_FLAT, 2 * EMBED_DIM), _imap2),
            pl.BlockSpec((EMBED_DIM,), _imap1),
        ],
        out_specs=pl.BlockSpec((ROWS, NUM_TOKENS, EMBED_DIM), _imap_out),
        out_shape=jax.ShapeDtypeStruct((BATCH, NUM_TOKENS, EMBED_DIM),
                                       jnp.float32),
        scratch_shapes=[
            pltpu.VMEM((ROWS * NUM_CHUNKS, CHUNK_FLAT), jnp.bfloat16),
            pltpu.VMEM((ROWS * NUM_CHUNKS, CHUNK_FLAT), jnp.bfloat16)],
    )(xf, w1cat, b1f, wcat, b2f)

    return tokens.astype(jnp.float64)
